# speedup vs baseline: 1.7002x; 1.7002x over previous
import functools

import jax
import jax.numpy as jnp
from jax import lax
from jax.experimental import pallas as pl
from jax.experimental.pallas import tpu as pltpu
from jax.experimental.pallas import tpu_sc as plsc


def _tc1_body(obj_ref, attr_ref, wa_ref, ba_ref, wt_ref, bt_ref, ws_ref,
              wo_ref, attr_out, ps_out, po_out):
    obj = obj_ref[...]
    attr_p = jnp.dot(attr_ref[...], wa_ref[...],
                     preferred_element_type=jnp.float32) + ba_ref[...]
    d = obj.shape[-1]
    h = (jnp.dot(obj, wt_ref[:d, :], preferred_element_type=jnp.float32)
         + jnp.dot(attr_p, wt_ref[d:, :], preferred_element_type=jnp.float32)
         + bt_ref[...])
    attr_out[...] = jnp.maximum(h, 0.0) + attr_p
    ps_out[...] = jnp.dot(obj, ws_ref[...], preferred_element_type=jnp.float32)
    po_out[...] = jnp.dot(obj, wo_ref[...], preferred_element_type=jnp.float32)


def _tc1(obj2, attr2, W_att, b_att, W_attr, b_attr, W_s, W_o):
    n, d = obj2.shape
    grid = 10
    blk = n // grid
    full = lambda i: (0, 0)
    row = lambda i: (i, 0)
    return pl.pallas_call(
        _tc1_body,
        grid=(grid,),
        in_specs=[
            pl.BlockSpec((blk, d), row),
            pl.BlockSpec((blk, 2 * d), row),
            pl.BlockSpec((2 * d, d), full),
            pl.BlockSpec((1, d), full),
            pl.BlockSpec((2 * d, d), full),
            pl.BlockSpec((1, d), full),
            pl.BlockSpec((d, d), full),
            pl.BlockSpec((d, d), full),
        ],
        out_specs=[
            pl.BlockSpec((blk, d), row),
            pl.BlockSpec((blk, d), row),
            pl.BlockSpec((blk, d), row),
        ],
        out_shape=[
            jax.ShapeDtypeStruct((n, d), jnp.float32),
            jax.ShapeDtypeStruct((n, d), jnp.float32),
            jax.ShapeDtypeStruct((n, d), jnp.float32),
        ],
    )(obj2, attr2, W_att, b_att, W_attr, b_attr, W_s, W_o)


_GRP = 128


def _sc_body(nblk, ps_hbm, po_hbm, sidx_hbm, oidx_hbm, out_hbm,
             sidx_v, oidx_v, a_v, b_v, sem):
    wid = lax.axis_index("s") * 2 + lax.axis_index("c")
    row0 = wid * nblk
    pltpu.sync_copy(sidx_hbm.at[pl.ds(row0, nblk)], sidx_v)
    pltpu.sync_copy(oidx_hbm.at[pl.ds(row0, nblk)], oidx_v)

    def blk(g, carry):
        c1 = pltpu.async_copy(ps_hbm.at[sidx_v.at[g]], a_v, sem)
        c2 = pltpu.async_copy(po_hbm.at[oidx_v.at[g]], b_v, sem)
        c1.wait()
        c2.wait()

        def row(r, c):
            for j in range(8):
                sl = pl.ds(j * 16, 16)
                a_v[r, sl] = a_v[r, sl] + b_v[r, sl]
            return c

        lax.fori_loop(0, _GRP, row, 0)
        pltpu.sync_copy(a_v, out_hbm.at[pl.ds((row0 + g) * _GRP, _GRP)])
        return carry

    lax.fori_loop(0, nblk, blk, 0)


def _sc_gather_sum(ps, po, sidx2, oidx2):
    n_grp = sidx2.shape[0]
    nw = 32
    nblk = n_grp // nw
    ep = n_grp * _GRP
    d = ps.shape[1]
    mesh = plsc.VectorSubcoreMesh(core_axis_name="c", subcore_axis_name="s")
    f = pl.kernel(
        functools.partial(_sc_body, nblk),
        out_type=jax.ShapeDtypeStruct((ep, d), jnp.float32),
        mesh=mesh,
        scratch_types=[
            pltpu.VMEM((nblk, _GRP), jnp.int32),
            pltpu.VMEM((nblk, _GRP), jnp.int32),
            pltpu.VMEM((_GRP, d), jnp.float32),
            pltpu.VMEM((_GRP, d), jnp.float32),
            pltpu.SemaphoreType.DMA,
        ],
    )
    return f(ps, po, sidx2, oidx2)


def _tc2_body(sum_ref, rela_ref, wr_ref, br_ref, mask_ref, out_ref):
    rela = rela_ref[...]
    h = (sum_ref[...]
         + jnp.dot(rela, wr_ref[...], preferred_element_type=jnp.float32)
         + br_ref[...])
    out_ref[...] = (jnp.maximum(h, 0.0) + rela) * mask_ref[...]


def _tc2(sum_g, rela2, W_r, b_rela, mask2):
    e, d = rela2.shape
    grid = 80
    blk = e // grid
    full = lambda i: (0, 0)
    row = lambda i: (i, 0)
    return pl.pallas_call(
        _tc2_body,
        grid=(grid,),
        in_specs=[
            pl.BlockSpec((blk, d), row),
            pl.BlockSpec((blk, d), row),
            pl.BlockSpec((d, d), full),
            pl.BlockSpec((1, d), full),
            pl.BlockSpec((blk, 1), row),
        ],
        out_specs=pl.BlockSpec((blk, d), row),
        out_shape=jax.ShapeDtypeStruct((e, d), jnp.float32),
    )(sum_g, rela2, W_r, b_rela, mask2)


def kernel(obj_vecs, attr_vecs, rela_vecs, edges, rela_masks,
           W_att, b_att, W_attr, b_attr, W_rela, b_rela):
    b, no, d = obj_vecs.shape
    nr = rela_vecs.shape[1]
    obj2 = obj_vecs.reshape(b * no, d)
    attr2 = attr_vecs.reshape(b * no, 2 * d)
    rela2 = rela_vecs.reshape(b * nr, d)

    new_attr2, ps, po = _tc1(obj2, attr2, W_att, b_att.reshape(1, d),
                             W_attr, b_attr.reshape(1, d),
                             W_rela[:d], W_rela[2 * d:])

    offs = jnp.arange(b, dtype=edges.dtype) * no
    e2 = (edges + offs[:, None, None]).reshape(-1, 2)
    e = b * nr
    n_grp = -(-e // (_GRP * 32)) * 32
    ep = n_grp * _GRP
    pad = jnp.zeros((ep - e,), e2.dtype)
    sidx2 = jnp.concatenate([e2[:, 0], pad]).reshape(n_grp, _GRP)
    oidx2 = jnp.concatenate([e2[:, 1], pad]).reshape(n_grp, _GRP)

    sum_g = _sc_gather_sum(ps, po, sidx2, oidx2)

    new_rela2 = _tc2(sum_g, rela2, W_rela[d:2 * d], b_rela.reshape(1, d),
                     rela_masks.reshape(b * nr, 1))

    return (obj_vecs,
            new_attr2.reshape(b, no, d),
            new_rela2.reshape(b, nr, d))


# speedup vs baseline: 2.0364x; 1.1977x over previous
import functools

import jax
import jax.numpy as jnp
from jax import lax
from jax.experimental import pallas as pl
from jax.experimental.pallas import tpu as pltpu
from jax.experimental.pallas import tpu_sc as plsc


def _tc1_body(obj_ref, attr_ref, wa_ref, ba_ref, wt_ref, bt_ref, ws_ref,
              wo_ref, attr_out, ps_out, po_out):
    obj = obj_ref[...]
    attr_p = jnp.dot(attr_ref[...], wa_ref[...],
                     preferred_element_type=jnp.float32) + ba_ref[...]
    d = obj.shape[-1]
    h = (jnp.dot(obj, wt_ref[:d, :], preferred_element_type=jnp.float32)
         + jnp.dot(attr_p, wt_ref[d:, :], preferred_element_type=jnp.float32)
         + bt_ref[...])
    attr_out[...] = jnp.maximum(h, 0.0) + attr_p
    ps_out[...] = jnp.dot(obj, ws_ref[...], preferred_element_type=jnp.float32)
    po_out[...] = jnp.dot(obj, wo_ref[...], preferred_element_type=jnp.float32)


def _tc1(obj2, attr2, W_att, b_att, W_attr, b_attr, W_s, W_o):
    n, d = obj2.shape
    grid = 10
    blk = n // grid
    full = lambda i: (0, 0)
    row = lambda i: (i, 0)
    return pl.pallas_call(
        _tc1_body,
        grid=(grid,),
        in_specs=[
            pl.BlockSpec((blk, d), row),
            pl.BlockSpec((blk, 2 * d), row),
            pl.BlockSpec((2 * d, d), full),
            pl.BlockSpec((1, d), full),
            pl.BlockSpec((2 * d, d), full),
            pl.BlockSpec((1, d), full),
            pl.BlockSpec((d, d), full),
            pl.BlockSpec((d, d), full),
        ],
        out_specs=[
            pl.BlockSpec((blk, d), row),
            pl.BlockSpec((blk, d), row),
            pl.BlockSpec((blk, d), row),
        ],
        out_shape=[
            jax.ShapeDtypeStruct((n, d), jnp.float32),
            jax.ShapeDtypeStruct((n, d), jnp.float32),
            jax.ShapeDtypeStruct((n, d), jnp.float32),
        ],
    )(obj2, attr2, W_att, b_att, W_attr, b_attr, W_s, W_o)


_GRP = 128
_BPW = 40


def _sc_body(d, ps_hbm, po_hbm, sidx_hbm, oidx_hbm, out_hbm,
             sidx_v, oidx_v, av0, bv0, av1, bv1, gs0, gs1, ws0, ws1):
    wid = lax.axis_index("s") * 2 + lax.axis_index("c")
    grp0 = wid * _BPW
    pltpu.sync_copy(sidx_hbm.at[pl.ds(grp0, _BPW)], sidx_v)
    pltpu.sync_copy(oidx_hbm.at[pl.ds(grp0, _BPW)], oidx_v)

    bufs = ((av0, bv0, gs0, ws0), (av1, bv1, gs1, ws1))

    def fire(gb, p):
        av, bv, gs, _ = bufs[p]
        pltpu.async_copy(ps_hbm.at[sidx_v.at[gb]], av, gs)
        pltpu.async_copy(po_hbm.at[oidx_v.at[gb]], bv, gs)

    def drain_gather(p):
        av, bv, gs, _ = bufs[p]
        pltpu.make_async_copy(ps_hbm.at[pl.ds(0, _GRP)], av, gs).wait()
        pltpu.make_async_copy(po_hbm.at[pl.ds(0, _GRP)], bv, gs).wait()

    def drain_wb(p):
        av, _, _, ws = bufs[p]
        pltpu.make_async_copy(av, out_hbm.at[pl.ds(0, _GRP)], ws).wait()

    fire(0, 0)

    def step(i, carry):
        for p in range(2):
            gb = 2 * i + p
            @pl.when(gb + 1 < _BPW)
            def _():
                @pl.when(gb >= 1)
                def _():
                    drain_wb(p ^ 1)
                fire(gb + 1, p ^ 1)

            av, bv, _, ws = bufs[p]
            drain_gather(p)

            def row(r, c):
                for j in range(d // 16):
                    sl = pl.ds(j * 16, 16)
                    av[r, sl] = av[r, sl] + bv[r, sl]
                return c

            lax.fori_loop(0, _GRP, row, 0)
            row_base = (grp0 + gb) * _GRP
            pltpu.async_copy(av, out_hbm.at[pl.ds(row_base, _GRP)], ws)
        return carry

    lax.fori_loop(0, _BPW // 2, step, 0)
    drain_wb(0)
    drain_wb(1)


def _sc_gather_sum(ps, po, sidx2, oidx2):
    n_grp = sidx2.shape[0]
    ep = n_grp * _GRP
    d = ps.shape[1]
    mesh = plsc.VectorSubcoreMesh(core_axis_name="c", subcore_axis_name="s")
    f = pl.kernel(
        functools.partial(_sc_body, d),
        out_type=jax.ShapeDtypeStruct((ep, d), jnp.float32),
        mesh=mesh,
        scratch_types=[
            pltpu.VMEM((_BPW, _GRP), jnp.int32),
            pltpu.VMEM((_BPW, _GRP), jnp.int32),
            pltpu.VMEM((_GRP, d), jnp.float32),
            pltpu.VMEM((_GRP, d), jnp.float32),
            pltpu.VMEM((_GRP, d), jnp.float32),
            pltpu.VMEM((_GRP, d), jnp.float32),
            pltpu.SemaphoreType.DMA,
            pltpu.SemaphoreType.DMA,
            pltpu.SemaphoreType.DMA,
            pltpu.SemaphoreType.DMA,
        ],
    )
    return f(ps, po, sidx2, oidx2)


def _tc2_body(sum_ref, rela_ref, wr_ref, br_ref, mask_ref, out_ref):
    rela = rela_ref[...]
    h = (sum_ref[...]
         + jnp.dot(rela, wr_ref[...], preferred_element_type=jnp.float32)
         + br_ref[...])
    out_ref[...] = (jnp.maximum(h, 0.0) + rela) * mask_ref[...]


def _tc2(sum_g, rela2, W_r, b_rela, mask2):
    e, d = rela2.shape
    grid = 80
    blk = e // grid
    full = lambda i: (0, 0)
    row = lambda i: (i, 0)
    return pl.pallas_call(
        _tc2_body,
        grid=(grid,),
        in_specs=[
            pl.BlockSpec((blk, d), row),
            pl.BlockSpec((blk, d), row),
            pl.BlockSpec((d, d), full),
            pl.BlockSpec((1, d), full),
            pl.BlockSpec((blk, 1), row),
        ],
        out_specs=pl.BlockSpec((blk, d), row),
        out_shape=jax.ShapeDtypeStruct((e, d), jnp.float32),
    )(sum_g, rela2, W_r, b_rela, mask2)


def kernel(obj_vecs, attr_vecs, rela_vecs, edges, rela_masks,
           W_att, b_att, W_attr, b_attr, W_rela, b_rela):
    b, no, d = obj_vecs.shape
    nr = rela_vecs.shape[1]
    obj2 = obj_vecs.reshape(b * no, d)
    attr2 = attr_vecs.reshape(b * no, 2 * d)
    rela2 = rela_vecs.reshape(b * nr, d)

    new_attr2, ps, po = _tc1(obj2, attr2, W_att, b_att.reshape(1, d),
                             W_attr, b_attr.reshape(1, d),
                             W_rela[:d], W_rela[2 * d:])

    offs = jnp.arange(b, dtype=edges.dtype) * no
    e2 = (edges + offs[:, None, None]).reshape(-1, 2)
    e = b * nr
    n_grp = -(-e // (_GRP * 32)) * 32
    ep = n_grp * _GRP
    pad = jnp.zeros((ep - e,), e2.dtype)
    sidx2 = jnp.concatenate([e2[:, 0], pad]).reshape(n_grp, _GRP)
    oidx2 = jnp.concatenate([e2[:, 1], pad]).reshape(n_grp, _GRP)

    sum_g = _sc_gather_sum(ps, po, sidx2, oidx2)

    new_rela2 = _tc2(sum_g, rela2, W_rela[d:2 * d], b_rela.reshape(1, d),
                     rela_masks.reshape(b * nr, 1))

    return (obj_vecs,
            new_attr2.reshape(b, no, d),
            new_rela2.reshape(b, nr, d))


# speedup vs baseline: 2.0932x; 1.0279x over previous
import functools

import jax
import jax.numpy as jnp
from jax import lax
from jax.experimental import pallas as pl
from jax.experimental.pallas import tpu as pltpu
from jax.experimental.pallas import tpu_sc as plsc


def _tc1_body(obj_ref, attr_ref, wa_ref, ba_ref, wt_ref, bt_ref, ws_ref,
              wo_ref, attr_out, ps_out, po_out):
    obj = obj_ref[...]
    attr_p = jnp.dot(attr_ref[...], wa_ref[...],
                     preferred_element_type=jnp.float32) + ba_ref[...]
    d = obj.shape[-1]
    h = (jnp.dot(obj, wt_ref[:d, :], preferred_element_type=jnp.float32)
         + jnp.dot(attr_p, wt_ref[d:, :], preferred_element_type=jnp.float32)
         + bt_ref[...])
    attr_out[...] = jnp.maximum(h, 0.0) + attr_p
    ps_out[...] = jnp.dot(obj, ws_ref[...], preferred_element_type=jnp.float32)
    po_out[...] = jnp.dot(obj, wo_ref[...], preferred_element_type=jnp.float32)


def _tc1(obj2, attr2, W_att, b_att, W_attr, b_attr, W_s, W_o):
    n, d = obj2.shape
    grid = 10
    blk = n // grid
    full = lambda i: (0, 0)
    row = lambda i: (i, 0)
    return pl.pallas_call(
        _tc1_body,
        grid=(grid,),
        in_specs=[
            pl.BlockSpec((blk, d), row),
            pl.BlockSpec((blk, 2 * d), row),
            pl.BlockSpec((2 * d, d), full),
            pl.BlockSpec((1, d), full),
            pl.BlockSpec((2 * d, d), full),
            pl.BlockSpec((1, d), full),
            pl.BlockSpec((d, d), full),
            pl.BlockSpec((d, d), full),
        ],
        out_specs=[
            pl.BlockSpec((blk, d), row),
            pl.BlockSpec((blk, d), row),
            pl.BlockSpec((blk, d), row),
        ],
        out_shape=[
            jax.ShapeDtypeStruct((n, d), jnp.float32),
            jax.ShapeDtypeStruct((n, d), jnp.float32),
            jax.ShapeDtypeStruct((n, d), jnp.float32),
        ],
    )(obj2, attr2, W_att, b_att, W_attr, b_attr, W_s, W_o)


_GRP = 128
_NW = 32


def _sc_body(d, nblk, ps_hbm, po_hbm, sidx_hbm, oidx_hbm, out_hbm,
             sidx_v, oidx_v, av0, bv0, av1, bv1, gs0, gs1, ws0, ws1):
    wid = lax.axis_index("s") * 2 + lax.axis_index("c")
    grp0 = wid * nblk
    pltpu.sync_copy(sidx_hbm.at[wid], sidx_v)
    pltpu.sync_copy(oidx_hbm.at[wid], oidx_v)

    bufs = ((av0, bv0, gs0, ws0), (av1, bv1, gs1, ws1))

    def fire(gb, p):
        av, bv, gs, _ = bufs[p]
        pltpu.async_copy(ps_hbm.at[sidx_v.at[gb]], av, gs)
        pltpu.async_copy(po_hbm.at[oidx_v.at[gb]], bv, gs)

    def drain_gather(p):
        av, bv, gs, _ = bufs[p]
        pltpu.make_async_copy(ps_hbm.at[pl.ds(0, _GRP)], av, gs).wait()
        pltpu.make_async_copy(po_hbm.at[pl.ds(0, _GRP)], bv, gs).wait()

    def drain_wb(p):
        av, _, _, ws = bufs[p]
        pltpu.make_async_copy(av, out_hbm.at[pl.ds(0, _GRP)], ws).wait()

    fire(0, 0)

    def step(i, carry):
        for p in range(2):
            gb = 2 * i + p
            @pl.when(gb + 1 < nblk)
            def _():
                @pl.when(gb >= 1)
                def _():
                    drain_wb(p ^ 1)
                fire(gb + 1, p ^ 1)

            av, bv, _, ws = bufs[p]
            drain_gather(p)

            def row(r, c):
                for j in range(d // 16):
                    sl = pl.ds(j * 16, 16)
                    av[r, sl] = av[r, sl] + bv[r, sl]
                return c

            lax.fori_loop(0, _GRP, row, 0)
            row_base = (grp0 + gb) * _GRP
            pltpu.async_copy(av, out_hbm.at[pl.ds(row_base, _GRP)], ws)
        return carry

    lax.fori_loop(0, nblk // 2, step, 0)
    drain_wb(0)
    drain_wb(1)


def _sc_gather_sum(ps, po, sidx2, oidx2):
    n_grp = sidx2.shape[0]
    nblk = n_grp // _NW
    ep = n_grp * _GRP
    d = ps.shape[1]
    sidx3 = sidx2.reshape(_NW, nblk, _GRP)
    oidx3 = oidx2.reshape(_NW, nblk, _GRP)
    mesh = plsc.VectorSubcoreMesh(core_axis_name="c", subcore_axis_name="s")
    f = pl.kernel(
        functools.partial(_sc_body, d, nblk),
        out_type=jax.ShapeDtypeStruct((ep, d), jnp.float32),
        mesh=mesh,
        scratch_types=[
            pltpu.VMEM((nblk, _GRP), jnp.int32),
            pltpu.VMEM((nblk, _GRP), jnp.int32),
            pltpu.VMEM((_GRP, d), jnp.float32),
            pltpu.VMEM((_GRP, d), jnp.float32),
            pltpu.VMEM((_GRP, d), jnp.float32),
            pltpu.VMEM((_GRP, d), jnp.float32),
            pltpu.SemaphoreType.DMA,
            pltpu.SemaphoreType.DMA,
            pltpu.SemaphoreType.DMA,
            pltpu.SemaphoreType.DMA,
        ],
    )
    return f(ps, po, sidx3, oidx3)


def _tc2_body(sum_ref, rela_ref, wr_ref, br_ref, mask_ref, out_ref):
    rela = rela_ref[...]
    h = (sum_ref[...]
         + jnp.dot(rela, wr_ref[...], preferred_element_type=jnp.float32)
         + br_ref[...])
    out_ref[...] = (jnp.maximum(h, 0.0) + rela) * mask_ref[...]


_TBLK = 1280


def _tc2(sum_h, rela2, W_r, b_rela, mask2, blk0, grid, prev=None):
    e, d = rela2.shape
    full = lambda i: (0, 0)
    row = lambda i: (i, 0)
    off = lambda i: (blk0 + i, 0)
    body = _tc2_body if prev is None else (
        lambda p_ref, *a: _tc2_body(*a))
    in_specs = [
        pl.BlockSpec((_TBLK, d), row),
        pl.BlockSpec((_TBLK, d), off),
        pl.BlockSpec((d, d), full),
        pl.BlockSpec((1, d), full),
        pl.BlockSpec((_TBLK, 1), off),
    ]
    args = (sum_h, rela2, W_r, b_rela, mask2)
    aliases = {}
    if prev is not None:
        in_specs = [pl.BlockSpec(memory_space=pltpu.MemorySpace.HBM)] + in_specs
        args = (prev,) + args
        aliases = {0: 0}
    return pl.pallas_call(
        body,
        grid=(grid,),
        in_specs=in_specs,
        out_specs=pl.BlockSpec((_TBLK, d), off),
        out_shape=jax.ShapeDtypeStruct((e, d), jnp.float32),
        input_output_aliases=aliases,
    )(*args)


def kernel(obj_vecs, attr_vecs, rela_vecs, edges, rela_masks,
           W_att, b_att, W_attr, b_attr, W_rela, b_rela):
    b, no, d = obj_vecs.shape
    nr = rela_vecs.shape[1]
    obj2 = obj_vecs.reshape(b * no, d)
    attr2 = attr_vecs.reshape(b * no, 2 * d)
    rela2 = rela_vecs.reshape(b * nr, d)

    new_attr2, ps, po = _tc1(obj2, attr2, W_att, b_att.reshape(1, d),
                             W_attr, b_attr.reshape(1, d),
                             W_rela[:d], W_rela[2 * d:])

    offs = jnp.arange(b, dtype=edges.dtype) * no
    e2 = (edges + offs[:, None, None]).reshape(-1, 2)
    e = b * nr
    n_grp = -(-e // (_GRP * _NW)) * _NW
    ep = n_grp * _GRP
    pad = jnp.zeros((ep - e,), e2.dtype)
    sidx2 = jnp.concatenate([e2[:, 0], pad]).reshape(n_grp, _GRP)
    oidx2 = jnp.concatenate([e2[:, 1], pad]).reshape(n_grp, _GRP)

    W_r = W_rela[d:2 * d]
    br = b_rela.reshape(1, d)
    mask2 = rela_masks.reshape(b * nr, 1)

    hg = n_grp // 2
    he = hg * _GRP
    new_rela2 = None
    for h in range(2):
        sum_h = _sc_gather_sum(ps, po,
                               sidx2[h * hg:(h + 1) * hg],
                               oidx2[h * hg:(h + 1) * hg])
        lo = h * he
        hi = min((h + 1) * he, e)
        new_rela2 = _tc2(sum_h, rela2, W_r, br, mask2,
                         blk0=lo // _TBLK, grid=(hi - lo) // _TBLK,
                         prev=new_rela2)

    return (obj_vecs,
            new_attr2.reshape(b, no, d),
            new_rela2.reshape(b, nr, d))
